# baseline (speedup 1.0000x reference)
import functools

import jax
import jax.numpy as jnp
from jax import lax
from jax.experimental import pallas as pl
from jax.experimental.pallas import tpu as pltpu
from jax.experimental.pallas import tpu_sc as plsc

NC = 2
NS = 16
CH = 128
ZCH = 64
HALVES = 2


def _sc_mesh():
    return plsc.VectorSubcoreMesh(
        core_axis_name="c", subcore_axis_name="s", num_cores=NC, num_subcores=NS
    )


def _make_deg(np_, d, nch):
    nr = np_ // NS

    @functools.partial(
        pl.kernel,
        mesh=_sc_mesh(),
        out_type=jax.ShapeDtypeStruct((NC, np_, d), jnp.float32),
        scratch_types=[
            pltpu.VMEM_SHARED((np_, d), jnp.float32),
            pltpu.VMEM((nch, CH), jnp.int32),
            pltpu.VMEM((CH, d), jnp.float32),
            pltpu.VMEM((ZCH, d), jnp.float32),
            pltpu.SemaphoreType.DMA,
        ],
    )
    def deg_kernel(dst_hbm, ones_hbm, zeros_hbm, out_hbm, acc, didx, ones_v, zeros_v,
                   ssem):
        c = lax.axis_index("c")
        s = lax.axis_index("s")
        pltpu.sync_copy(zeros_hbm, zeros_v)
        for k in range(nr // ZCH):
            pltpu.sync_copy(zeros_v, acc.at[pl.ds(s * nr + k * ZCH, ZCH)])
        pltpu.sync_copy(ones_hbm, ones_v)
        pltpu.sync_copy(dst_hbm.at[c, s], didx)
        plsc.subcore_barrier()

        def fire(j, carry):
            pltpu.async_copy(ones_v, acc.at[didx.at[j]], ssem, add=True)
            return carry

        lax.fori_loop(0, nch, fire, 0)

        def drain(j, carry):
            pltpu.make_async_copy(ones_v, acc.at[didx.at[0]], ssem).wait()
            return carry

        lax.fori_loop(0, nch, drain, 0)
        plsc.subcore_barrier()
        pltpu.sync_copy(acc.at[pl.ds(s * nr, nr)], out_hbm.at[c, pl.ds(s * nr, nr)])

    return deg_kernel


def _make_spmm(n, np_, d, nhch):
    nr = np_ // NS
    assert nhch % 2 == 0

    @functools.partial(
        pl.kernel,
        mesh=_sc_mesh(),
        out_type=jax.ShapeDtypeStruct((NC, np_, d), jnp.float32),
        scratch_types=[
            pltpu.VMEM_SHARED((np_, d), jnp.float32),
            pltpu.VMEM((nhch, CH), jnp.int32),
            pltpu.VMEM((nhch, CH), jnp.int32),
            pltpu.VMEM((2, CH, d), jnp.float32),
            pltpu.SemaphoreType.DMA,
            pltpu.SemaphoreType.DMA,
        ],
    )
    def spmm_kernel(h_hbm, src_hbm, dst_hbm, out_hbm,
                    acc, sidx, didx, rows_v, g0, g1):
        c = lax.axis_index("c")
        s = lax.axis_index("s")

        def zstore(t, carry):
            i = t // (d // 16)
            jj = lax.rem(t, d // 16)
            rows_v[0, i, pl.ds(jj * 16, 16)] = jnp.zeros((16,), jnp.float32)
            return carry

        lax.fori_loop(0, ZCH * (d // 16), zstore, 0)
        for k in range(nr // ZCH):
            pltpu.sync_copy(rows_v.at[0, pl.ds(0, ZCH)],
                            acc.at[pl.ds(s * nr + k * ZCH, ZCH)])
        plsc.subcore_barrier()

        for half in range(HALVES):
            pltpu.sync_copy(src_hbm.at[c, s, half], sidx)
            pltpu.sync_copy(dst_hbm.at[c, s, half], didx)
            pltpu.async_copy(h_hbm.at[sidx.at[0]], rows_v.at[0], g0)

            def chunk2(j2, carry):
                j = 2 * j2
                pltpu.async_copy(h_hbm.at[sidx.at[j + 1]], rows_v.at[1], g1)
                pltpu.make_async_copy(h_hbm.at[sidx.at[j]], rows_v.at[0], g0).wait()
                pltpu.sync_copy(rows_v.at[0], acc.at[didx.at[j]], add=True)

                @pl.when(j2 + 1 < nhch // 2)
                def _():
                    pltpu.async_copy(h_hbm.at[sidx.at[j + 2]], rows_v.at[0], g0)

                pltpu.make_async_copy(
                    h_hbm.at[sidx.at[j + 1]], rows_v.at[1], g1).wait()
                pltpu.sync_copy(rows_v.at[1], acc.at[didx.at[j + 1]], add=True)
                return carry

            lax.fori_loop(0, nhch // 2, chunk2, 0)

        plsc.subcore_barrier()
        pltpu.sync_copy(acc.at[pl.ds(s * nr, nr)], out_hbm.at[c, pl.ds(s * nr, nr)])

    return spmm_kernel


def _tc_pre(dego, x, w1):
    def body(dego_ref, x_ref, w1_ref, h1_ref, dinv_ref):
        n = x_ref.shape[0]
        deg = dego_ref[0, :n, 0:1] + dego_ref[1, :n, 0:1] + 1.0
        dinv = lax.rsqrt(jnp.maximum(deg, 1.0))
        dinv_ref[...] = dinv
        xs = x_ref[...] * dinv
        h1_ref[...] = jnp.dot(xs, w1_ref[...], preferred_element_type=jnp.float32)

    n = x.shape[0]
    return pl.pallas_call(
        body,
        out_shape=(
            jax.ShapeDtypeStruct((n, w1.shape[1]), jnp.float32),
            jax.ShapeDtypeStruct((n, 1), jnp.float32),
        ),
    )(dego, x, w1)


def _tc_mid(p, hp, dinv, b, gamma, beta, wn):
    def body(p_ref, hp_ref, dinv_ref, b_ref, g_ref, be_ref, wn_ref, hn_ref, r_ref):
        n = hp_ref.shape[0]
        dinv = dinv_ref[...]
        agg = (p_ref[0, :n] + p_ref[1, :n] + hp_ref[...]) * dinv + b_ref[...]
        m = jnp.mean(agg, axis=0, keepdims=True)
        v = jnp.mean((agg - m) ** 2, axis=0, keepdims=True)
        z = (agg - m) * lax.rsqrt(v + 1e-5) * g_ref[...] + be_ref[...]
        r = jnp.maximum(z, 0.0)
        r_ref[...] = r
        hn_ref[...] = jnp.dot(r * dinv, wn_ref[...], preferred_element_type=jnp.float32)

    n, d = hp.shape
    return pl.pallas_call(
        body,
        out_shape=(
            jax.ShapeDtypeStruct((n, wn.shape[1]), jnp.float32),
            jax.ShapeDtypeStruct((n, d), jnp.float32),
        ),
    )(p, hp, dinv, b.reshape(1, -1), gamma.reshape(1, -1), beta.reshape(1, -1), wn)


def _tc_fin(p, hp, dinv, b):
    def body(p_ref, hp_ref, dinv_ref, b_ref, out_ref):
        n = hp_ref.shape[0]
        out_ref[...] = (
            p_ref[0, :n] + p_ref[1, :n] + hp_ref[...]
        ) * dinv_ref[...] + b_ref[...]

    n, d = hp.shape
    return pl.pallas_call(
        body,
        out_shape=jax.ShapeDtypeStruct((n, d), jnp.float32),
    )(p, hp, dinv, b.reshape(1, -1))


def kernel(x, edge_index, W1, b1, gamma1, beta1, W2, b2, gamma2, beta2, W3, b3):
    n, d = x.shape
    e = edge_index.shape[1]
    ntile = NC * NS
    np_ = ((n + NS * ZCH - 1) // (NS * ZCH)) * (NS * ZCH)
    chunk_mult = CH * HALVES * 2
    per_tile = -(-e // (ntile * chunk_mult)) * chunk_mult
    e_pad = per_tile * ntile
    nch = per_tile // CH
    nhch = nch // HALVES
    assert nch % HALVES == 0 and nhch % 2 == 0

    pad = e_pad - e
    pad_dst = (n + jnp.arange(pad, dtype=jnp.int32) % (np_ - n)).astype(jnp.int32)
    src = jnp.concatenate([edge_index[0], jnp.zeros((pad,), jnp.int32)])
    dst = jnp.concatenate([edge_index[1], pad_dst])
    src_r = src.reshape(NC, NS, HALVES, nhch, CH)
    dst_r = dst.reshape(NC, NS, HALVES, nhch, CH)
    dst_deg = dst.reshape(NC, NS, nch, CH)
    onerows = jnp.ones((CH, d), jnp.float32)
    zrows = jnp.zeros((ZCH, d), jnp.float32)

    deg_k = _make_deg(np_, d, nch)
    spmm_k = _make_spmm(n, np_, d, nhch)

    dego = deg_k(dst_deg, onerows, zrows)
    h1, dinv = _tc_pre(dego, x, W1)

    p1 = spmm_k(h1, src_r, dst_r)
    h2, _ = _tc_mid(p1, h1, dinv, b1, gamma1, beta1, W2)

    p2 = spmm_k(h2, src_r, dst_r)
    h3, emb = _tc_mid(p2, h2, dinv, b2, gamma2, beta2, W3)

    p3 = spmm_k(h3, src_r, dst_r)
    hc = _tc_fin(p3, h3, dinv, b3)
    return (emb, hc)


# speedup vs baseline: 3.0677x; 3.0677x over previous
import functools

import jax
import jax.numpy as jnp
from jax import lax
from jax.experimental import pallas as pl
from jax.experimental.pallas import tpu as pltpu
from jax.experimental.pallas import tpu_sc as plsc

NC = 2
NS = 16
CH = 128
ZCH = 64
HALVES = 2


def _sc_mesh():
    return plsc.VectorSubcoreMesh(
        core_axis_name="c", subcore_axis_name="s", num_cores=NC, num_subcores=NS
    )


def _make_deg(np_, d, nch):
    nr = np_ // NS

    @functools.partial(
        pl.kernel,
        mesh=_sc_mesh(),
        out_type=jax.ShapeDtypeStruct((NC, np_, d), jnp.float32),
        scratch_types=[
            pltpu.VMEM_SHARED((np_, d), jnp.float32),
            pltpu.VMEM((nch, CH), jnp.int32),
            pltpu.VMEM((CH, d), jnp.float32),
            pltpu.VMEM((ZCH, d), jnp.float32),
            pltpu.SemaphoreType.DMA,
        ],
    )
    def deg_kernel(dst_hbm, ones_hbm, zeros_hbm, out_hbm, acc, didx, ones_v, zeros_v,
                   ssem):
        c = lax.axis_index("c")
        s = lax.axis_index("s")
        pltpu.sync_copy(zeros_hbm, zeros_v)
        for k in range(nr // ZCH):
            pltpu.sync_copy(zeros_v, acc.at[pl.ds(s * nr + k * ZCH, ZCH)])
        pltpu.sync_copy(ones_hbm, ones_v)
        pltpu.sync_copy(dst_hbm.at[c, s], didx)
        plsc.subcore_barrier()

        def fire(j, carry):
            pltpu.async_copy(ones_v, acc.at[didx.at[j]], ssem, add=True)
            return carry

        lax.fori_loop(0, nch, fire, 0)

        def drain(j, carry):
            pltpu.make_async_copy(ones_v, acc.at[didx.at[0]], ssem).wait()
            return carry

        lax.fori_loop(0, nch, drain, 0)
        plsc.subcore_barrier()
        pltpu.sync_copy(acc.at[pl.ds(s * nr, nr)], out_hbm.at[c, pl.ds(s * nr, nr)])

    return deg_kernel


def _make_spmm(n, np_, d, nhch):
    nr = np_ // NS
    assert nhch % 2 == 0

    @functools.partial(
        pl.kernel,
        mesh=_sc_mesh(),
        out_type=jax.ShapeDtypeStruct((NC, np_, d), jnp.float32),
        scratch_types=[
            pltpu.VMEM_SHARED((np_, d), jnp.float32),
            pltpu.VMEM((nhch, CH), jnp.int32),
            pltpu.VMEM((nhch, CH), jnp.int32),
            pltpu.VMEM((2, CH, d), jnp.float32),
            pltpu.SemaphoreType.DMA,
            pltpu.SemaphoreType.DMA,
        ],
    )
    def spmm_kernel(h_hbm, src_hbm, dst_hbm, out_hbm,
                    acc, sidx, didx, rows_v, g0, g1):
        c = lax.axis_index("c")
        s = lax.axis_index("s")

        def zstore(t, carry):
            i = t // (d // 16)
            jj = lax.rem(t, d // 16)
            rows_v[0, i, pl.ds(jj * 16, 16)] = jnp.zeros((16,), jnp.float32)
            return carry

        lax.fori_loop(0, ZCH * (d // 16), zstore, 0)
        for k in range(nr // ZCH):
            pltpu.sync_copy(rows_v.at[0, pl.ds(0, ZCH)],
                            acc.at[pl.ds(s * nr + k * ZCH, ZCH)])
        plsc.subcore_barrier()

        for half in range(HALVES):
            pltpu.sync_copy(src_hbm.at[c, s, half], sidx)
            pltpu.sync_copy(dst_hbm.at[c, s, half], didx)
            pltpu.async_copy(h_hbm.at[sidx.at[0]], rows_v.at[0], g0)

            def chunk2(j2, carry):
                j = 2 * j2
                pltpu.async_copy(h_hbm.at[sidx.at[j + 1]], rows_v.at[1], g1)
                pltpu.make_async_copy(h_hbm.at[sidx.at[j]], rows_v.at[0], g0).wait()
                pltpu.sync_copy(rows_v.at[0], acc.at[didx.at[j]], add=True)

                @pl.when(j2 + 1 < nhch // 2)
                def _():
                    pltpu.async_copy(h_hbm.at[sidx.at[j + 2]], rows_v.at[0], g0)

                pltpu.make_async_copy(
                    h_hbm.at[sidx.at[j + 1]], rows_v.at[1], g1).wait()
                pltpu.sync_copy(rows_v.at[1], acc.at[didx.at[j + 1]], add=True)
                return carry

            lax.fori_loop(0, nhch // 2, chunk2, 0)

        plsc.subcore_barrier()
        pltpu.sync_copy(acc.at[pl.ds(s * nr, nr)], out_hbm.at[c, pl.ds(s * nr, nr)])

    return spmm_kernel


def _tc_pre(dego, x, w1):
    def body(dego_ref, x_ref, w1_ref, h1_ref, dinv_ref):
        n = x_ref.shape[0]
        deg = dego_ref[0, :n, 0:1] + dego_ref[1, :n, 0:1] + 1.0
        dinv = lax.rsqrt(jnp.maximum(deg, 1.0))
        dinv_ref[...] = dinv
        xs = x_ref[...] * dinv
        h1_ref[...] = jnp.dot(xs, w1_ref[...], preferred_element_type=jnp.float32)

    n = x.shape[0]
    return pl.pallas_call(
        body,
        out_shape=(
            jax.ShapeDtypeStruct((n, w1.shape[1]), jnp.float32),
            jax.ShapeDtypeStruct((n, 1), jnp.float32),
        ),
    )(dego, x, w1)


def _tc_mid(p, hp, dinv, b, gamma, beta, wn):
    def body(p_ref, hp_ref, dinv_ref, b_ref, g_ref, be_ref, wn_ref, hn_ref, r_ref):
        n = hp_ref.shape[0]
        dinv = dinv_ref[...]
        agg = (p_ref[0, :n] + p_ref[1, :n] + hp_ref[...]) * dinv + b_ref[...]
        m = jnp.mean(agg, axis=0, keepdims=True)
        v = jnp.mean((agg - m) ** 2, axis=0, keepdims=True)
        z = (agg - m) * lax.rsqrt(v + 1e-5) * g_ref[...] + be_ref[...]
        r = jnp.maximum(z, 0.0)
        r_ref[...] = r
        hn_ref[...] = jnp.dot(r * dinv, wn_ref[...], preferred_element_type=jnp.float32)

    n, d = hp.shape
    return pl.pallas_call(
        body,
        out_shape=(
            jax.ShapeDtypeStruct((n, wn.shape[1]), jnp.float32),
            jax.ShapeDtypeStruct((n, d), jnp.float32),
        ),
    )(p, hp, dinv, b.reshape(1, -1), gamma.reshape(1, -1), beta.reshape(1, -1), wn)


def _tc_fin(p, hp, dinv, b):
    def body(p_ref, hp_ref, dinv_ref, b_ref, out_ref):
        n = hp_ref.shape[0]
        out_ref[...] = (
            p_ref[0, :n] + p_ref[1, :n] + hp_ref[...]
        ) * dinv_ref[...] + b_ref[...]

    n, d = hp.shape
    return pl.pallas_call(
        body,
        out_shape=jax.ShapeDtypeStruct((n, d), jnp.float32),
    )(p, hp, dinv, b.reshape(1, -1))


def kernel(x, edge_index, W1, b1, gamma1, beta1, W2, b2, gamma2, beta2, W3, b3):
    n, d = x.shape
    e = edge_index.shape[1]
    ntile = NC * NS
    np_ = ((n + NS * ZCH - 1) // (NS * ZCH)) * (NS * ZCH)
    chunk_mult = CH * HALVES * 2
    per_tile = -(-e // (ntile * chunk_mult)) * chunk_mult
    e_pad = per_tile * ntile
    nch = per_tile // CH
    nhch = nch // HALVES
    assert nch % HALVES == 0 and nhch % 2 == 0

    pad = e_pad - e
    per_tile_real = e // ntile
    pad_per_tile = per_tile - per_tile_real
    assert per_tile_real * ntile == e
    pad_src = (jnp.arange(pad_per_tile, dtype=jnp.int32) % n).astype(jnp.int32)
    pad_src = jnp.broadcast_to(pad_src, (ntile, pad_per_tile))
    pad_dst = (n + jnp.arange(pad_per_tile, dtype=jnp.int32) % (np_ - n)).astype(
        jnp.int32)
    pad_dst = jnp.broadcast_to(pad_dst, (ntile, pad_per_tile))
    src = jnp.concatenate(
        [edge_index[0].reshape(ntile, per_tile_real), pad_src], axis=1)
    dst = jnp.concatenate(
        [edge_index[1].reshape(ntile, per_tile_real), pad_dst], axis=1)
    src_r = src.reshape(NC, NS, HALVES, nhch, CH)
    dst_r = dst.reshape(NC, NS, HALVES, nhch, CH)
    dst_deg = dst.reshape(NC, NS, nch, CH)
    onerows = jnp.ones((CH, d), jnp.float32)
    zrows = jnp.zeros((ZCH, d), jnp.float32)

    deg_k = _make_deg(np_, d, nch)
    spmm_k = _make_spmm(n, np_, d, nhch)

    dego = deg_k(dst_deg, onerows, zrows)
    h1, dinv = _tc_pre(dego, x, W1)

    p1 = spmm_k(h1, src_r, dst_r)
    h2, _ = _tc_mid(p1, h1, dinv, b1, gamma1, beta1, W2)

    p2 = spmm_k(h2, src_r, dst_r)
    h3, emb = _tc_mid(p2, h2, dinv, b2, gamma2, beta2, W3)

    p3 = spmm_k(h3, src_r, dst_r)
    hc = _tc_fin(p3, h3, dinv, b3)
    return (emb, hc)


# speedup vs baseline: 3.6313x; 1.1837x over previous
import functools

import jax
import jax.numpy as jnp
from jax import lax
from jax.experimental import pallas as pl
from jax.experimental.pallas import tpu as pltpu
from jax.experimental.pallas import tpu_sc as plsc

NC = 2
NS = 16
CH = 128
ZCH = 32
HALVES = 2


def _sc_mesh():
    return plsc.VectorSubcoreMesh(
        core_axis_name="c", subcore_axis_name="s", num_cores=NC, num_subcores=NS
    )


def _make_deg(np_, per_tile):
    @functools.partial(
        pl.kernel,
        mesh=_sc_mesh(),
        compiler_params=pltpu.CompilerParams(needs_layout_passes=False),
        out_type=jax.ShapeDtypeStruct((NC * NS, np_), jnp.float32),
        scratch_types=[
            pltpu.VMEM((np_,), jnp.float32),
            pltpu.VMEM((per_tile,), jnp.int32),
        ],
    )
    def deg_kernel(dst_hbm, out_hbm, dloc, didx):
        c = lax.axis_index("c")
        s = lax.axis_index("s")
        gid = c * NS + s
        pltpu.sync_copy(dst_hbm.at[c, s], didx)

        def zero(t, carry):
            dloc[pl.ds(t * 16, 16)] = jnp.zeros((16,), jnp.float32)
            return carry

        lax.fori_loop(0, np_ // 16, zero, 0)
        ones = jnp.ones((16,), jnp.float32)

        def scat(t, carry):
            idxv = didx[pl.ds(t * 16, 16)]
            plsc.addupdate_scatter(dloc, [idxv], ones)
            return carry

        lax.fori_loop(0, per_tile // 16, scat, 0)
        pltpu.sync_copy(dloc, out_hbm.at[gid])

    return deg_kernel


def _make_spmm(n, np_, d, nhch):
    nr = np_ // NS
    assert nhch % 2 == 0

    @functools.partial(
        pl.kernel,
        mesh=_sc_mesh(),
        out_type=jax.ShapeDtypeStruct((NC, np_, d), jnp.float32),
        scratch_types=[
            pltpu.VMEM_SHARED((np_, d), jnp.float32),
            pltpu.VMEM((nhch, CH), jnp.int32),
            pltpu.VMEM((nhch, CH), jnp.int32),
            pltpu.VMEM((2, CH, d), jnp.float32),
            pltpu.VMEM((ZCH, d), jnp.float32),
            pltpu.SemaphoreType.DMA,
            pltpu.SemaphoreType.DMA,
            pltpu.SemaphoreType.DMA,
        ],
    )
    def spmm_kernel(h_hbm, src_hbm, dst_hbm, out_hbm,
                    acc, sidx, didx, rows_v, zbuf, g0, g1, zsem):
        c = lax.axis_index("c")
        s = lax.axis_index("s")

        pltpu.sync_copy(src_hbm.at[c, s, 0], sidx)
        pltpu.sync_copy(dst_hbm.at[c, s, 0], didx)
        pltpu.async_copy(h_hbm.at[sidx.at[0]], rows_v.at[0], g0)
        pltpu.async_copy(h_hbm.at[sidx.at[1]], rows_v.at[1], g1)

        def zstore(t, carry):
            i = t // (d // 16)
            jj = lax.rem(t, d // 16)
            zbuf[i, pl.ds(jj * 16, 16)] = jnp.zeros((16,), jnp.float32)
            return carry

        lax.fori_loop(0, ZCH * (d // 16), zstore, 0)
        for k in range(nr // ZCH):
            pltpu.async_copy(zbuf, acc.at[pl.ds(s * nr + k * ZCH, ZCH)], zsem)
        for k in range(nr // ZCH):
            pltpu.make_async_copy(zbuf, acc.at[pl.ds(0, ZCH)], zsem).wait()
        plsc.subcore_barrier()

        for half in range(HALVES):
            if half > 0:
                pltpu.sync_copy(src_hbm.at[c, s, half], sidx)
                pltpu.sync_copy(dst_hbm.at[c, s, half], didx)
                pltpu.async_copy(h_hbm.at[sidx.at[0]], rows_v.at[0], g0)
                pltpu.async_copy(h_hbm.at[sidx.at[1]], rows_v.at[1], g1)

            def chunk2(j2, carry):
                j = 2 * j2
                pltpu.make_async_copy(h_hbm.at[sidx.at[j]], rows_v.at[0], g0).wait()
                pltpu.sync_copy(rows_v.at[0], acc.at[didx.at[j]], add=True)

                @pl.when(j2 + 1 < nhch // 2)
                def _():
                    pltpu.async_copy(h_hbm.at[sidx.at[j + 2]], rows_v.at[0], g0)

                pltpu.make_async_copy(
                    h_hbm.at[sidx.at[j + 1]], rows_v.at[1], g1).wait()
                pltpu.sync_copy(rows_v.at[1], acc.at[didx.at[j + 1]], add=True)

                @pl.when(j2 + 1 < nhch // 2)
                def _():
                    pltpu.async_copy(h_hbm.at[sidx.at[j + 3]], rows_v.at[1], g1)

                return carry

            lax.fori_loop(0, nhch // 2, chunk2, 0)

        plsc.subcore_barrier()
        pltpu.sync_copy(acc.at[pl.ds(s * nr, nr)], out_hbm.at[c, pl.ds(s * nr, nr)])

    return spmm_kernel


def _tc_mm(x, w):
    def body(x_ref, w_ref, o_ref):
        o_ref[...] = jnp.dot(x_ref[...], w_ref[...],
                             preferred_element_type=jnp.float32)

    n = x.shape[0]
    return pl.pallas_call(
        body,
        out_shape=jax.ShapeDtypeStruct((n, w.shape[1]), jnp.float32),
    )(x, w)


def _tc_pre(dego, xw):
    def body(dego_ref, xw_ref, h1_ref, dinv_ref):
        n = xw_ref.shape[0]
        deg = jnp.sum(dego_ref[...], axis=0)[:n, None] + 1.0
        dinv = lax.rsqrt(jnp.maximum(deg, 1.0))
        dinv_ref[...] = dinv
        h1_ref[...] = xw_ref[...] * dinv

    n, dh = xw.shape
    return pl.pallas_call(
        body,
        out_shape=(
            jax.ShapeDtypeStruct((n, dh), jnp.float32),
            jax.ShapeDtypeStruct((n, 1), jnp.float32),
        ),
    )(dego, xw)


def _tc_mid(p, hp, dinv, b, gamma, beta, wn):
    def body(p_ref, hp_ref, dinv_ref, b_ref, g_ref, be_ref, wn_ref, hn_ref, r_ref):
        n = hp_ref.shape[0]
        dinv = dinv_ref[...]
        agg = (p_ref[0, :n] + p_ref[1, :n] + hp_ref[...]) * dinv + b_ref[...]
        m = jnp.mean(agg, axis=0, keepdims=True)
        v = jnp.mean((agg - m) ** 2, axis=0, keepdims=True)
        z = (agg - m) * lax.rsqrt(v + 1e-5) * g_ref[...] + be_ref[...]
        r = jnp.maximum(z, 0.0)
        r_ref[...] = r
        hn_ref[...] = jnp.dot(r * dinv, wn_ref[...], preferred_element_type=jnp.float32)

    n, d = hp.shape
    return pl.pallas_call(
        body,
        out_shape=(
            jax.ShapeDtypeStruct((n, wn.shape[1]), jnp.float32),
            jax.ShapeDtypeStruct((n, d), jnp.float32),
        ),
    )(p, hp, dinv, b.reshape(1, -1), gamma.reshape(1, -1), beta.reshape(1, -1), wn)


def _tc_fin(p, hp, dinv, b):
    def body(p_ref, hp_ref, dinv_ref, b_ref, out_ref):
        n = hp_ref.shape[0]
        out_ref[...] = (
            p_ref[0, :n] + p_ref[1, :n] + hp_ref[...]
        ) * dinv_ref[...] + b_ref[...]

    n, d = hp.shape
    return pl.pallas_call(
        body,
        out_shape=jax.ShapeDtypeStruct((n, d), jnp.float32),
    )(p, hp, dinv, b.reshape(1, -1))


def kernel(x, edge_index, W1, b1, gamma1, beta1, W2, b2, gamma2, beta2, W3, b3):
    n, d = x.shape
    e = edge_index.shape[1]
    ntile = NC * NS
    np_ = ((n + NS * ZCH - 1) // (NS * ZCH)) * (NS * ZCH)
    chunk_mult = CH * HALVES * 2
    per_tile = -(-e // (ntile * chunk_mult)) * chunk_mult
    e_pad = per_tile * ntile
    nch = per_tile // CH
    nhch = nch // HALVES
    assert nch % HALVES == 0 and nhch % 2 == 0

    pad = e_pad - e
    per_tile_real = e // ntile
    pad_per_tile = per_tile - per_tile_real
    assert per_tile_real * ntile == e
    pad_src = (jnp.arange(pad_per_tile, dtype=jnp.int32) % n).astype(jnp.int32)
    pad_src = jnp.broadcast_to(pad_src, (ntile, pad_per_tile))
    pad_dst = (n + jnp.arange(pad_per_tile, dtype=jnp.int32) % (np_ - n)).astype(
        jnp.int32)
    pad_dst = jnp.broadcast_to(pad_dst, (ntile, pad_per_tile))
    src = jnp.concatenate(
        [edge_index[0].reshape(ntile, per_tile_real), pad_src], axis=1)
    dst = jnp.concatenate(
        [edge_index[1].reshape(ntile, per_tile_real), pad_dst], axis=1)
    src_r = src.reshape(NC, NS, HALVES, nhch, CH)
    dst_r = dst.reshape(NC, NS, HALVES, nhch, CH)
    dst_deg = dst.reshape(NC, NS, per_tile)

    deg_k = _make_deg(np_, per_tile)
    spmm_k = _make_spmm(n, np_, d, nhch)

    xw = _tc_mm(x, W1)
    dego = deg_k(dst_deg)
    h1, dinv = _tc_pre(dego, xw)

    p1 = spmm_k(h1, src_r, dst_r)
    h2, _ = _tc_mid(p1, h1, dinv, b1, gamma1, beta1, W2)

    p2 = spmm_k(h2, src_r, dst_r)
    h3, emb = _tc_mid(p2, h2, dinv, b2, gamma2, beta2, W3)

    p3 = spmm_k(h3, src_r, dst_r)
    hc = _tc_fin(p3, h3, dinv, b3)
    return (emb, hc)
